# Initial kernel scaffold; baseline (speedup 1.0000x reference)
import jax
import jax.numpy as jnp
from jax.experimental import pallas as pl


def kernel(x, edge_index, edge_attr, batch, feat_table, edge_tables, W1, b1, W2, b2, gamma, beta, Wf1, bf1, Wf2, bf2):
    raise NotImplementedError("write your pallas kernel here")


# speedup vs baseline: 5.3520x; 5.3520x over previous
import functools

import jax
import jax.numpy as jnp
from jax import lax
from jax.experimental import pallas as pl
from jax.experimental.pallas import tpu as pltpu
from jax.experimental.pallas import tpu_sc as plsc

EMB = 128
NLAYER = 5
NN = 10000
NE = 320000
NG = 64

NC = 2
NS = 16
NTILE = NC * NS
CHUNK = 128
NCH = -(-NE // (NTILE * CHUNK))
NE_PAD = NTILE * NCH * CHUNK
AGG_ROWS = 10112
ROWS_PER_SUB = AGG_ROWS // NS
DUMP_ROW = NN
F_ROWS = 4 * NN

def _sc_agg_body(f_hbm, gidx_hbm, dst_hbm, zeros_hbm, out_hbm,
                 gidx_v, dst_v, rows_v, agg_sh, sem):
    cid = lax.axis_index("c")
    sid = lax.axis_index("s")
    wid = cid * NS + sid
    pltpu.sync_copy(gidx_hbm.at[wid], gidx_v)
    pltpu.sync_copy(dst_hbm.at[wid], dst_v)
    row0 = sid * ROWS_PER_SUB
    pltpu.sync_copy(zeros_hbm.at[pl.ds(row0, ROWS_PER_SUB)],
                    agg_sh.at[pl.ds(row0, ROWS_PER_SUB)])
    plsc.subcore_barrier()

    @pl.loop(0, NCH)
    def _chunk(j):
        pltpu.async_copy(f_hbm.at[gidx_v.at[j]], rows_v, sem).wait()
        pltpu.sync_copy(rows_v, agg_sh.at[dst_v.at[j]], add=True)

    plsc.subcore_barrier()
    pltpu.sync_copy(agg_sh.at[pl.ds(row0, ROWS_PER_SUB)],
                    out_hbm.at[cid, pl.ds(row0, ROWS_PER_SUB)])


@functools.lru_cache(maxsize=None)
def _get_sc_agg():
    mesh = plsc.VectorSubcoreMesh(
        core_axis_name="c", subcore_axis_name="s",
        num_cores=NC, num_subcores=NS)
    return pl.kernel(
        _sc_agg_body,
        out_type=jax.ShapeDtypeStruct((NC, AGG_ROWS, EMB), jnp.float32),
        mesh=mesh,
        scratch_types=[
            pltpu.VMEM((NCH, CHUNK), jnp.int32),
            pltpu.VMEM((NCH, CHUNK), jnp.int32),
            pltpu.VMEM((CHUNK, EMB), jnp.float32),
            pltpu.VMEM_SHARED((AGG_ROWS, EMB), jnp.float32),
            pltpu.SemaphoreType.DMA,
        ],
    )


def _build_f(f_ref, h, t_ref):
    for a in range(4):
        f_ref[a * NN:(a + 1) * NN, :] = jnp.maximum(h + t_ref[a:a + 1, :], 0.0)


def _enc_body(x_ref, src_ref, ea_ref, tab_ref, t0_ref, h_ref, f_ref, gidx_ref):
    iota = lax.broadcasted_iota(jnp.int32, (NN, 32), 1)
    oh = (x_ref[...] == iota).astype(jnp.float32)
    h = jnp.dot(oh, tab_ref[...], preferred_element_type=jnp.float32)
    h_ref[...] = h
    _build_f(f_ref, h, t0_ref)
    gidx_ref[...] = ea_ref[...] * NN + src_ref[...]


_tc_encode = pl.pallas_call(
    _enc_body,
    out_shape=(
        jax.ShapeDtypeStruct((NN, EMB), jnp.float32),
        jax.ShapeDtypeStruct((F_ROWS, EMB), jnp.float32),
        jax.ShapeDtypeStruct((NE // 128, 128), jnp.int32),
    ),
)


def _mlp_bn(h, agg_ref, w1_ref, b1_ref, w2_ref, b2_ref, g_ref, be_ref):
    z = h + agg_ref[0, :NN, :] + agg_ref[1, :NN, :]
    z = jnp.maximum(
        jnp.dot(z, w1_ref[...], preferred_element_type=jnp.float32)
        + b1_ref[...], 0.0)
    z = jnp.dot(z, w2_ref[...], preferred_element_type=jnp.float32) + b2_ref[...]
    mu = jnp.mean(z, axis=0, keepdims=True)
    zc = z - mu
    var = jnp.mean(zc * zc, axis=0, keepdims=True)
    z = zc * lax.rsqrt(var + 1e-5) * g_ref[...] + be_ref[...]
    return jnp.maximum(z, 0.0) + h


def _layer_body(h_ref, agg_ref, w1_ref, b1_ref, w2_ref, b2_ref, g_ref, be_ref,
                t_ref, ho_ref, f_ref):
    hn = _mlp_bn(h_ref[...], agg_ref, w1_ref, b1_ref, w2_ref, b2_ref, g_ref,
                 be_ref)
    ho_ref[...] = hn
    _build_f(f_ref, hn, t_ref)


_tc_layer = pl.pallas_call(
    _layer_body,
    out_shape=(
        jax.ShapeDtypeStruct((NN, EMB), jnp.float32),
        jax.ShapeDtypeStruct((F_ROWS, EMB), jnp.float32),
    ),
)


def _final_body(h_ref, agg_ref, w1_ref, b1_ref, w2_ref, b2_ref, g_ref, be_ref,
                batch_ref, wf1_ref, bf1_ref, wf2_ref, bf2_ref, out_ref):
    hn = _mlp_bn(h_ref[...], agg_ref, w1_ref, b1_ref, w2_ref, b2_ref, g_ref,
                 be_ref)
    iota = lax.broadcasted_iota(jnp.int32, (NN, NG), 1)
    oh = (batch_ref[...] == iota).astype(jnp.float32)
    pooled = lax.dot_general(oh, hn, (((0,), (0,)), ((), ())),
                             preferred_element_type=jnp.float32)
    p = jnp.maximum(
        jnp.dot(pooled, wf1_ref[...], preferred_element_type=jnp.float32)
        + bf1_ref[...], 0.0)
    out_ref[...] = (
        jnp.dot(p, wf2_ref[...], preferred_element_type=jnp.float32)
        + bf2_ref[...])


_tc_final = pl.pallas_call(
    _final_body,
    out_shape=jax.ShapeDtypeStruct((NG, 1), jnp.float32),
)


def kernel(x, edge_index, edge_attr, batch, feat_table, edge_tables,
           W1, b1, W2, b2, gamma, beta, Wf1, bf1, Wf2, bf2):
    x = x.astype(jnp.int32)
    src = edge_index[0].astype(jnp.int32)
    dst = edge_index[1].astype(jnp.int32)
    ea = edge_attr.astype(jnp.int32)
    batch = batch.astype(jnp.int32)

    tab32 = jnp.zeros((32, EMB), jnp.float32).at[:21].set(feat_table)
    h, f, gidx2d = _tc_encode(
        x.reshape(NN, 1), src.reshape(NE // 128, 128),
        ea.reshape(NE // 128, 128), tab32, edge_tables[0])

    pad = NE_PAD - NE
    gidx_t = jnp.concatenate(
        [gidx2d.reshape(NE), jnp.zeros((pad,), jnp.int32)]
    ).reshape(NTILE, NCH, CHUNK)
    dst_t = jnp.concatenate(
        [dst, jnp.full((pad,), DUMP_ROW, jnp.int32)]
    ).reshape(NTILE, NCH, CHUNK)
    zeros_agg = jnp.zeros((AGG_ROWS, EMB), jnp.float32)

    sc_agg = _get_sc_agg()
    for l in range(NLAYER):
        agg = sc_agg(f, gidx_t, dst_t, zeros_agg)
        if l < NLAYER - 1:
            h, f = _tc_layer(
                h, agg, W1[l], b1[l].reshape(1, EMB), W2[l],
                b2[l].reshape(1, EMB), gamma[l].reshape(1, EMB),
                beta[l].reshape(1, EMB), edge_tables[l + 1])
        else:
            out = _tc_final(
                h, agg, W1[l], b1[l].reshape(1, EMB), W2[l],
                b2[l].reshape(1, EMB), gamma[l].reshape(1, EMB),
                beta[l].reshape(1, EMB), batch.reshape(NN, 1),
                Wf1, bf1.reshape(1, 2 * EMB), Wf2, bf2.reshape(1, 1))
    return out
